# baseline (speedup 1.0000x reference)
import functools
import math

import jax
import jax.numpy as jnp
from jax import lax
from jax.experimental import pallas as pl
from jax.experimental.pallas import tpu as pltpu
from jax.experimental.pallas import tpu_sc as plsc

D = 64
L = 16
NC, NS = 2, 16
NW = NC * NS
CHUNK = 128
NBUF = 4
SCALE = math.sqrt(D)


def _body(idx_hbm, table_hbm, out_hbm, idx_v, rows_v, gsem):
    n_chunks = idx_hbm.shape[0]
    g_per_w = n_chunks // NW
    wid = lax.axis_index("s") * NC + lax.axis_index("c")
    base_g = wid * g_per_w

    pltpu.sync_copy(idx_hbm.at[pl.ds(base_g, g_per_w)], idx_v)

    def start(j, b):
        pltpu.async_copy(table_hbm.at[idx_v.at[j]], rows_v.at[b], gsem.at[b])

    for b in range(NBUF):
        start(b, b)

    def scale_buf(buf):
        def row(r, _):
            for c in range(D // L):
                sl = (r, pl.ds(c * L, L))
                buf[sl] = buf[sl] * SCALE
            return 0
        lax.fori_loop(0, CHUNK, row, 0)

    n_grp = g_per_w // NBUF

    def grp_body(g, _):
        for b in range(NBUF):
            j = g * NBUF + b
            pltpu.make_async_copy(
                table_hbm.at[idx_v.at[j]], rows_v.at[b], gsem.at[b]
            ).wait()
            scale_buf(rows_v.at[b])
            pltpu.sync_copy(
                rows_v.at[b], out_hbm.at[pl.ds((base_g + j) * CHUNK, CHUNK)]
            )
            nxt = j + NBUF

            @pl.when(nxt < g_per_w)
            def _():
                start(nxt, b)
        return 0

    lax.fori_loop(0, n_grp, grp_body, 0)


@functools.lru_cache(maxsize=None)
def _build(n_chunks):
    mesh = plsc.VectorSubcoreMesh(core_axis_name="c", subcore_axis_name="s")
    g_per_w = n_chunks // NW
    return pl.kernel(
        _body,
        out_type=jax.ShapeDtypeStruct((n_chunks * CHUNK, D), jnp.float32),
        mesh=mesh,
        scratch_types=[
            pltpu.VMEM((g_per_w, CHUNK), jnp.int32),
            pltpu.VMEM((NBUF, CHUNK, D), jnp.float32),
            pltpu.SemaphoreType.DMA((NBUF,)),
        ],
        compiler_params=pltpu.CompilerParams(use_tc_tiling_on_sc=False),
    )


def kernel(x, table):
    orig_shape = x.shape
    flat = x.reshape(-1).astype(jnp.int32)
    b = flat.shape[0]
    assert b % (NW * CHUNK * NBUF) == 0, b
    idx2d = flat.reshape(b // CHUNK, CHUNK)
    out = _build(b // CHUNK)(idx2d, table)
    return out.reshape(*orig_shape, D)
